# baseline (speedup 1.0000x reference)
import functools

import jax
import jax.numpy as jnp
from jax import lax
from jax.experimental import pallas as pl
from jax.experimental.pallas import tpu as pltpu
from jax.experimental.pallas import tpu_sc as plsc

EMBED = 1024
TEXT_END = 32000
IMG_OFFSET = 32000
L = 16


def _project_codebook(codebook, w):
    vq_vocab, vq_embed = codebook.shape
    bm = 512

    def body(cb_ref, w_ref, o_ref):
        o_ref[...] = lax.dot_general(
            cb_ref[...], w_ref[...],
            dimension_numbers=(((1,), (1,)), ((), ())),
            preferred_element_type=jnp.float32)

    return pl.pallas_call(
        body,
        grid=(vq_vocab // bm,),
        in_specs=[
            pl.BlockSpec((bm, vq_embed), lambda i: (i, 0)),
            pl.BlockSpec((EMBED, vq_embed), lambda i: (0, 0)),
        ],
        out_specs=pl.BlockSpec((bm, EMBED), lambda i: (i, 0)),
        out_shape=jax.ShapeDtypeStruct((vq_vocab, EMBED), jnp.float32),
    )(codebook, w)


@functools.cache
def _make_mover(n_tokens, is_text, CH, nb):
    info = plsc.get_sparse_core_info()
    nw = info.num_cores * info.num_subcores
    tpw = n_tokens // nw
    assert n_tokens % nw == 0 and tpw % L == 0
    mesh = plsc.VectorSubcoreMesh(core_axis_name="c", subcore_axis_name="s")

    @functools.partial(
        pl.kernel,
        mesh=mesh,
        out_type=(),
        compiler_params=pltpu.CompilerParams(needs_layout_passes=False),
        scratch_types=[
            pltpu.VMEM((tpw,), jnp.int32),
            pltpu.VMEM((tpw,), jnp.int32),
            pltpu.VMEM((tpw,), jnp.int32),
        ] + [pltpu.VMEM((CH, EMBED), jnp.float32)] * nb + [
            pltpu.SemaphoreType.DMA,
            pltpu.SemaphoreType.DMA,
        ],
    )
    def k(x_hbm, table_hbm, out_hbm, x_v, cidx, cpos, *rest):
        bufs = rest[:nb]
        sem_g, sem_s = rest[nb], rest[nb + 1]
        wid = lax.axis_index("s") * info.num_cores + lax.axis_index("c")
        base = wid * tpw
        row_len = x_hbm.shape[1]
        wpr = row_len // tpw
        pltpu.sync_copy(
            x_hbm.at[wid // wpr, pl.ds((wid % wpr) * tpw, tpw)], x_v)

        lanes = lax.iota(jnp.int32, L)

        def compact(j, n):
            xv = x_v[pl.ds(j * L, L)]
            if is_text:
                m = xv < TEXT_END
                val = xv
            else:
                m = xv >= TEXT_END
                val = xv - IMG_OFFSET
            m32 = m.astype(jnp.int32)
            incl = plsc.cumsum(m32)
            slot = n + (incl - m32)
            pos = base + j * L + lanes
            plsc.store_scatter(cidx, [slot], val, mask=m)
            plsc.store_scatter(cpos, [slot], pos, mask=m)
            return n + incl[L - 1]

        n = lax.fori_loop(0, tpw // L, compact, jnp.int32(0))

        zeros16 = jnp.zeros((L,), jnp.int32)
        idx0 = plsc.load_gather(cidx, [zeros16])
        pos0 = plsc.load_gather(cpos, [zeros16])
        n8 = (n + 7) & -8
        pad_end = jnp.where(n8 < CH, jnp.int32(CH), n8)
        for kk in range(max(CH // L, 1)):
            slot = n + kk * L + lanes
            m = slot < pad_end
            plsc.store_scatter(cidx, [slot], idx0, mask=m)
            plsc.store_scatter(cpos, [slot], pos0, mask=m)
        ne = jnp.where(n > 0, pad_end, jnp.int32(0))

        nch = (ne + CH - 1) // CH
        last = jnp.maximum(ne - CH, 0)

        def start_gather(c, b):
            s = pl.multiple_of(jnp.minimum(c * CH, last), 8)
            pltpu.make_async_copy(
                table_hbm.at[cidx.at[pl.ds(s, CH)]], b, sem_g).start()

        def wait_gather(b):
            pltpu.make_async_copy(
                table_hbm.at[cidx.at[pl.ds(0, CH)]], b, sem_g).wait()

        def start_scatter(c, b):
            s = pl.multiple_of(jnp.minimum(c * CH, last), 8)
            pltpu.make_async_copy(
                b, out_hbm.at[cpos.at[pl.ds(s, CH)]], sem_s).start()

        def wait_scatter(b):
            pltpu.make_async_copy(
                b, out_hbm.at[cpos.at[pl.ds(0, CH)]], sem_s).wait()

        for b in range(nb):
            @pl.when(b < nch)
            def _(b=b):
                start_gather(b, bufs[b])

        def group(p, c):
            g0 = p * nb
            for b in range(nb):
                j = g0 + b

                @pl.when(j < nch)
                def _(j=j, b=b):
                    wait_gather(bufs[b])
                    start_scatter(j, bufs[b])

                    @pl.when(j + nb < nch)
                    def _():
                        wait_scatter(bufs[b])
                        start_gather(j + nb, bufs[b])
            return c

        lax.fori_loop(0, (nch + nb - 1) // nb, group, 0)
        for b in range(nb):
            @pl.when(b < nch)
            def _(b=b):
                wait_scatter(bufs[b])

    return k


def kernel(x, token_embedding, vqgan_codebook, vqgan_proj_W):
    pc = _project_codebook(vqgan_codebook, vqgan_proj_W)
    n_tokens = x.shape[0] * x.shape[1]
    out_ref = jax.new_ref(jax.lax.empty((n_tokens, EMBED), jnp.float32))
    _make_mover(n_tokens, True, 16, 7)(x, token_embedding, out_ref)
    _make_mover(n_tokens, False, 8, 14)(x, pc, out_ref)
    return out_ref[...].reshape(x.shape + (EMBED,))


# speedup vs baseline: 1.0065x; 1.0065x over previous
import functools

import jax
import jax.numpy as jnp
from jax import lax
from jax.experimental import pallas as pl
from jax.experimental.pallas import tpu as pltpu
from jax.experimental.pallas import tpu_sc as plsc

EMBED = 1024
TEXT_END = 32000
IMG_OFFSET = 32000
L = 16


def _project_codebook(codebook, w):
    vq_vocab, vq_embed = codebook.shape
    bm = 512

    def body(cb_ref, w_ref, o_ref):
        o_ref[...] = lax.dot_general(
            cb_ref[...], w_ref[...],
            dimension_numbers=(((1,), (1,)), ((), ())),
            preferred_element_type=jnp.float32)

    return pl.pallas_call(
        body,
        grid=(vq_vocab // bm,),
        in_specs=[
            pl.BlockSpec((bm, vq_embed), lambda i: (i, 0)),
            pl.BlockSpec((EMBED, vq_embed), lambda i: (0, 0)),
        ],
        out_specs=pl.BlockSpec((bm, EMBED), lambda i: (i, 0)),
        out_shape=jax.ShapeDtypeStruct((vq_vocab, EMBED), jnp.float32),
    )(codebook, w)


@functools.cache
def _make_mover(n_tokens, is_text, CH, nb):
    info = plsc.get_sparse_core_info()
    nw = info.num_cores * info.num_subcores
    tpw = n_tokens // nw
    assert n_tokens % nw == 0 and tpw % L == 0
    mesh = plsc.VectorSubcoreMesh(core_axis_name="c", subcore_axis_name="s")

    @functools.partial(
        pl.kernel,
        mesh=mesh,
        out_type=(),
        compiler_params=pltpu.CompilerParams(needs_layout_passes=False),
        scratch_types=[
            pltpu.VMEM((tpw,), jnp.int32),
            pltpu.VMEM((tpw,), jnp.int32),
            pltpu.VMEM((tpw,), jnp.int32),
        ] + [pltpu.VMEM((CH, EMBED), jnp.float32)] * nb + [
            pltpu.SemaphoreType.DMA,
            pltpu.SemaphoreType.DMA,
        ],
    )
    def k(x_hbm, table_hbm, out_hbm, x_v, cidx, cpos, *rest):
        bufs = rest[:nb]
        sem_g, sem_s = rest[nb], rest[nb + 1]
        wid = lax.axis_index("s") * info.num_cores + lax.axis_index("c")
        base = wid * tpw
        row_len = x_hbm.shape[1]
        wpr = row_len // tpw
        pltpu.sync_copy(
            x_hbm.at[wid // wpr, pl.ds((wid % wpr) * tpw, tpw)], x_v)

        lanes = lax.iota(jnp.int32, L)

        def compact(j, n):
            xv = x_v[pl.ds(j * L, L)]
            if is_text:
                m = xv < TEXT_END
                val = xv
            else:
                m = xv >= TEXT_END
                val = xv - IMG_OFFSET
            m32 = m.astype(jnp.int32)
            incl = plsc.cumsum(m32)
            slot = n + (incl - m32)
            pos = base + j * L + lanes
            plsc.store_scatter(cidx, [slot], val, mask=m)
            plsc.store_scatter(cpos, [slot], pos, mask=m)
            return n + incl[L - 1]

        n = lax.fori_loop(0, tpw // L, compact, jnp.int32(0))

        zeros16 = jnp.zeros((L,), jnp.int32)
        idx0 = plsc.load_gather(cidx, [zeros16])
        pos0 = plsc.load_gather(cpos, [zeros16])
        n8 = (n + 7) & -8
        pad_end = jnp.where(n8 < CH, jnp.int32(CH), n8)
        for kk in range(max(CH // L, 1)):
            slot = n + kk * L + lanes
            m = slot < pad_end
            plsc.store_scatter(cidx, [slot], idx0, mask=m)
            plsc.store_scatter(cpos, [slot], pos0, mask=m)
        ne = jnp.where(n > 0, pad_end, jnp.int32(0))

        nch = (ne + CH - 1) // CH
        last = jnp.maximum(ne - CH, 0)

        def start_gather(c, b):
            s = pl.multiple_of(jnp.minimum(c * CH, last), 8)
            pltpu.make_async_copy(
                table_hbm.at[cidx.at[pl.ds(s, CH)]], b, sem_g).start()

        def wait_gather(b):
            pltpu.make_async_copy(
                table_hbm.at[cidx.at[pl.ds(0, CH)]], b, sem_g).wait()

        def start_scatter(c, b):
            s = pl.multiple_of(jnp.minimum(c * CH, last), 8)
            pltpu.make_async_copy(
                b, out_hbm.at[cpos.at[pl.ds(s, CH)]], sem_s).start()

        def wait_scatter(b):
            pltpu.make_async_copy(
                b, out_hbm.at[cpos.at[pl.ds(0, CH)]], sem_s).wait()

        for b in range(nb):
            @pl.when(b < nch)
            def _(b=b):
                start_gather(b, bufs[b])

        def group(p, c):
            g0 = p * nb
            for b in range(nb):
                j = g0 + b

                @pl.when(j < nch)
                def _(j=j, b=b):
                    wait_gather(bufs[b])
                    start_scatter(j, bufs[b])

                    @pl.when(j + nb < nch)
                    def _():
                        wait_scatter(bufs[b])
                        start_gather(j + nb, bufs[b])
            return c

        lax.fori_loop(0, (nch + nb - 1) // nb, group, 0)
        for b in range(nb):
            @pl.when(b < nch)
            def _(b=b):
                wait_scatter(bufs[b])

    return k


def kernel(x, token_embedding, vqgan_codebook, vqgan_proj_W):
    pc = _project_codebook(vqgan_codebook, vqgan_proj_W)
    n_tokens = x.shape[0] * x.shape[1]
    out_ref = jax.new_ref(jax.lax.empty((n_tokens, EMBED), jnp.float32))
    _make_mover(n_tokens, True, 16, 7)(x, token_embedding, out_ref)
    _make_mover(n_tokens, False, 16, 7)(x, pc, out_ref)
    return out_ref[...].reshape(x.shape + (EMBED,))


# speedup vs baseline: 1.0073x; 1.0007x over previous
import functools

import jax
import jax.numpy as jnp
from jax import lax
from jax.experimental import pallas as pl
from jax.experimental.pallas import tpu as pltpu
from jax.experimental.pallas import tpu_sc as plsc

EMBED = 1024
TEXT_END = 32000
IMG_OFFSET = 32000
L = 16


def _project_codebook(codebook, w):
    vq_vocab, vq_embed = codebook.shape
    bm = 512

    def body(cb_ref, w_ref, o_ref):
        o_ref[...] = lax.dot_general(
            cb_ref[...], w_ref[...],
            dimension_numbers=(((1,), (1,)), ((), ())),
            preferred_element_type=jnp.float32)

    return pl.pallas_call(
        body,
        grid=(vq_vocab // bm,),
        in_specs=[
            pl.BlockSpec((bm, vq_embed), lambda i: (i, 0)),
            pl.BlockSpec((EMBED, vq_embed), lambda i: (0, 0)),
        ],
        out_specs=pl.BlockSpec((bm, EMBED), lambda i: (i, 0)),
        out_shape=jax.ShapeDtypeStruct((vq_vocab, EMBED), jnp.float32),
    )(codebook, w)


@functools.cache
def _make_mover(n_tokens, is_text, CH, nb):
    info = plsc.get_sparse_core_info()
    nw = info.num_cores * info.num_subcores
    tpw = n_tokens // nw
    assert n_tokens % nw == 0 and tpw % L == 0
    mesh = plsc.VectorSubcoreMesh(core_axis_name="c", subcore_axis_name="s")

    @functools.partial(
        pl.kernel,
        mesh=mesh,
        out_type=(),
        compiler_params=pltpu.CompilerParams(needs_layout_passes=False),
        scratch_types=[
            pltpu.VMEM((tpw,), jnp.int32),
            pltpu.VMEM((tpw,), jnp.int32),
            pltpu.VMEM((tpw,), jnp.int32),
        ] + [pltpu.VMEM((CH, EMBED), jnp.float32)] * nb + [
            pltpu.SemaphoreType.DMA,
            pltpu.SemaphoreType.DMA,
        ],
    )
    def k(x_hbm, table_hbm, out_hbm, x_v, cidx, cpos, *rest):
        bufs = rest[:nb]
        sem_g, sem_s = rest[nb], rest[nb + 1]
        wid = lax.axis_index("s") * info.num_cores + lax.axis_index("c")
        base = wid * tpw
        row_len = x_hbm.shape[1]
        wpr = row_len // tpw
        pltpu.sync_copy(
            x_hbm.at[wid // wpr, pl.ds((wid % wpr) * tpw, tpw)], x_v)

        lanes = lax.iota(jnp.int32, L)

        def compact(j, n):
            xv = x_v[pl.ds(j * L, L)]
            if is_text:
                m = xv < TEXT_END
                val = xv
            else:
                m = xv >= TEXT_END
                val = xv - IMG_OFFSET
            m32 = m.astype(jnp.int32)
            incl = plsc.cumsum(m32)
            slot = n + (incl - m32)
            pos = base + j * L + lanes
            plsc.store_scatter(cidx, [slot], val, mask=m)
            plsc.store_scatter(cpos, [slot], pos, mask=m)
            return n + incl[L - 1]

        n = lax.fori_loop(0, tpw // L, compact, jnp.int32(0))

        zeros16 = jnp.zeros((L,), jnp.int32)
        idx0 = plsc.load_gather(cidx, [zeros16])
        pos0 = plsc.load_gather(cpos, [zeros16])
        n8 = (n + 7) & -8
        pad_end = jnp.where(n8 < CH, jnp.int32(CH), n8)
        for kk in range(max(CH // L, 1)):
            slot = n + kk * L + lanes
            m = slot < pad_end
            plsc.store_scatter(cidx, [slot], idx0, mask=m)
            plsc.store_scatter(cpos, [slot], pos0, mask=m)
        ne = jnp.where(n > 0, pad_end, jnp.int32(0))

        nch = (ne + CH - 1) // CH
        last = jnp.maximum(ne - CH, 0)

        def start_gather(c, b):
            s = pl.multiple_of(jnp.minimum(c * CH, last), 8)
            pltpu.make_async_copy(
                table_hbm.at[cidx.at[pl.ds(s, CH)]], b, sem_g).start()

        def wait_gather(b):
            pltpu.make_async_copy(
                table_hbm.at[cidx.at[pl.ds(0, CH)]], b, sem_g).wait()

        def start_scatter(c, b):
            s = pl.multiple_of(jnp.minimum(c * CH, last), 8)
            pltpu.make_async_copy(
                b, out_hbm.at[cpos.at[pl.ds(s, CH)]], sem_s).start()

        def wait_scatter(b):
            pltpu.make_async_copy(
                b, out_hbm.at[cpos.at[pl.ds(0, CH)]], sem_s).wait()

        for b in range(nb):
            @pl.when(b < nch)
            def _(b=b):
                start_gather(b, bufs[b])

        def group(p, c):
            g0 = p * nb
            for b in range(nb):
                j = g0 + b

                @pl.when(j < nch)
                def _(j=j, b=b):
                    wait_gather(bufs[b])
                    start_scatter(j, bufs[b])

                    @pl.when(j + nb < nch)
                    def _():
                        wait_scatter(bufs[b])
                        start_gather(j + nb, bufs[b])
            return c

        lax.fori_loop(0, (nch + nb - 1) // nb, group, 0)
        for b in range(nb):
            @pl.when(b < nch)
            def _(b=b):
                wait_scatter(bufs[b])

    return k


def kernel(x, token_embedding, vqgan_codebook, vqgan_proj_W):
    pc = _project_codebook(vqgan_codebook, vqgan_proj_W)
    n_tokens = x.shape[0] * x.shape[1]
    out_ref = jax.new_ref(jax.lax.empty((n_tokens, EMBED), jnp.float32))
    _make_mover(n_tokens, True, 24, 5)(x, token_embedding, out_ref)
    _make_mover(n_tokens, False, 16, 7)(x, pc, out_ref)
    return out_ref[...].reshape(x.shape + (EMBED,))


# speedup vs baseline: 1.0090x; 1.0017x over previous
import functools

import jax
import jax.numpy as jnp
from jax import lax
from jax.experimental import pallas as pl
from jax.experimental.pallas import tpu as pltpu
from jax.experimental.pallas import tpu_sc as plsc

EMBED = 1024
TEXT_END = 32000
IMG_OFFSET = 32000
L = 16


def _project_codebook(codebook, w):
    vq_vocab, vq_embed = codebook.shape
    bm = 512

    def body(cb_ref, w_ref, o_ref):
        o_ref[...] = lax.dot_general(
            cb_ref[...], w_ref[...],
            dimension_numbers=(((1,), (1,)), ((), ())),
            preferred_element_type=jnp.float32)

    return pl.pallas_call(
        body,
        grid=(vq_vocab // bm,),
        in_specs=[
            pl.BlockSpec((bm, vq_embed), lambda i: (i, 0)),
            pl.BlockSpec((EMBED, vq_embed), lambda i: (0, 0)),
        ],
        out_specs=pl.BlockSpec((bm, EMBED), lambda i: (i, 0)),
        out_shape=jax.ShapeDtypeStruct((vq_vocab, EMBED), jnp.float32),
    )(codebook, w)


@functools.cache
def _make_mover(n_tokens, is_text, CH, nb):
    info = plsc.get_sparse_core_info()
    nw = info.num_cores * info.num_subcores
    tpw = n_tokens // nw
    assert n_tokens % nw == 0 and tpw % L == 0
    mesh = plsc.VectorSubcoreMesh(core_axis_name="c", subcore_axis_name="s")

    @functools.partial(
        pl.kernel,
        mesh=mesh,
        out_type=(),
        compiler_params=pltpu.CompilerParams(needs_layout_passes=False),
        scratch_types=[
            pltpu.VMEM((tpw,), jnp.int32),
            pltpu.VMEM((tpw,), jnp.int32),
            pltpu.VMEM((tpw,), jnp.int32),
        ] + [pltpu.VMEM((CH, EMBED), jnp.float32)] * nb + [
            pltpu.SemaphoreType.DMA,
            pltpu.SemaphoreType.DMA,
        ],
    )
    def k(x_hbm, table_hbm, out_hbm, x_v, cidx, cpos, *rest):
        bufs = rest[:nb]
        sem_g, sem_s = rest[nb], rest[nb + 1]
        wid = lax.axis_index("s") * info.num_cores + lax.axis_index("c")
        base = wid * tpw
        row_len = x_hbm.shape[1]
        wpr = row_len // tpw
        pltpu.sync_copy(
            x_hbm.at[wid // wpr, pl.ds((wid % wpr) * tpw, tpw)], x_v)

        lanes = lax.iota(jnp.int32, L)

        def compact(j, n):
            xv = x_v[pl.ds(j * L, L)]
            if is_text:
                m = xv < TEXT_END
                val = xv
            else:
                m = xv >= TEXT_END
                val = xv - IMG_OFFSET
            m32 = m.astype(jnp.int32)
            incl = plsc.cumsum(m32)
            slot = n + (incl - m32)
            pos = base + j * L + lanes
            plsc.store_scatter(cidx, [slot], val, mask=m)
            plsc.store_scatter(cpos, [slot], pos, mask=m)
            return n + incl[L - 1]

        n_q = lax.fori_loop(0, tpw // L // 4, compact, jnp.int32(0))
        nch_early = n_q // CH

        def start_gather_early(c, b):
            pltpu.make_async_copy(
                table_hbm.at[cidx.at[pl.ds(c * CH, CH)]], b, sem_g).start()

        for b in range(nb):
            @pl.when(b < nch_early)
            def _(b=b):
                start_gather_early(b, bufs[b])

        n = lax.fori_loop(tpw // L // 4, tpw // L, compact, n_q)

        zeros16 = jnp.zeros((L,), jnp.int32)
        idx0 = plsc.load_gather(cidx, [zeros16])
        pos0 = plsc.load_gather(cpos, [zeros16])
        n8 = (n + 7) & -8
        pad_end = jnp.where(n8 < CH, jnp.int32(CH), n8)
        for kk in range(max(CH // L, 1)):
            slot = n + kk * L + lanes
            m = slot < pad_end
            plsc.store_scatter(cidx, [slot], idx0, mask=m)
            plsc.store_scatter(cpos, [slot], pos0, mask=m)
        ne = jnp.where(n > 0, pad_end, jnp.int32(0))

        nch = (ne + CH - 1) // CH
        last = jnp.maximum(ne - CH, 0)

        def start_gather(c, b):
            s = pl.multiple_of(jnp.minimum(c * CH, last), 8)
            pltpu.make_async_copy(
                table_hbm.at[cidx.at[pl.ds(s, CH)]], b, sem_g).start()

        def wait_gather(b):
            pltpu.make_async_copy(
                table_hbm.at[cidx.at[pl.ds(0, CH)]], b, sem_g).wait()

        def start_scatter(c, b):
            s = pl.multiple_of(jnp.minimum(c * CH, last), 8)
            pltpu.make_async_copy(
                b, out_hbm.at[cpos.at[pl.ds(s, CH)]], sem_s).start()

        def wait_scatter(b):
            pltpu.make_async_copy(
                b, out_hbm.at[cpos.at[pl.ds(0, CH)]], sem_s).wait()

        for b in range(nb):
            @pl.when(jnp.logical_and(b >= nch_early, b < nch))
            def _(b=b):
                start_gather(b, bufs[b])

        def group(p, c):
            g0 = p * nb
            for b in range(nb):
                j = g0 + b

                @pl.when(j < nch)
                def _(j=j, b=b):
                    wait_gather(bufs[b])
                    start_scatter(j, bufs[b])

                    @pl.when(j + nb < nch)
                    def _():
                        wait_scatter(bufs[b])
                        start_gather(j + nb, bufs[b])
            return c

        lax.fori_loop(0, (nch + nb - 1) // nb, group, 0)
        for b in range(nb):
            @pl.when(b < nch)
            def _(b=b):
                wait_scatter(bufs[b])

    return k


def kernel(x, token_embedding, vqgan_codebook, vqgan_proj_W):
    pc = _project_codebook(vqgan_codebook, vqgan_proj_W)
    n_tokens = x.shape[0] * x.shape[1]
    out_ref = jax.new_ref(jax.lax.empty((n_tokens, EMBED), jnp.float32))
    _make_mover(n_tokens, True, 24, 5)(x, token_embedding, out_ref)
    _make_mover(n_tokens, False, 16, 7)(x, pc, out_ref)
    return out_ref[...].reshape(x.shape + (EMBED,))
